# baseline (speedup 1.0000x reference)
import functools

import jax
import jax.numpy as jnp
from jax import lax
from jax.experimental import pallas as pl
from jax.experimental.pallas import tpu as pltpu
from jax.experimental.pallas import tpu_sc as plsc

_N = 100000
_D = 128
_TOTAL = _N * _D
_NW = 32
_PER_W = _TOTAL // _NW
_CHUNK = 16000
_NCHUNK = _PER_W // _CHUNK

_mesh = plsc.VectorSubcoreMesh(core_axis_name="c", subcore_axis_name="s")


@functools.partial(
    pl.kernel,
    mesh=_mesh,
    out_type=jax.ShapeDtypeStruct((_TOTAL,), jnp.float32),
    scratch_types=[
        pltpu.VMEM((48,), jnp.float32),
        pltpu.VMEM((_CHUNK,), jnp.float32),
        pltpu.VMEM((_CHUNK,), jnp.float32),
        pltpu.VMEM((_CHUNK,), jnp.float32),
        pltpu.VMEM((_CHUNK,), jnp.float32),
    ],
)
def _sc_agg(e0_hbm, e1_hbm, e2_hbm, c_hbm, out_hbm, c_v, b0, b1, b2, ob):
    wid = lax.axis_index("s") * 2 + lax.axis_index("c")
    base = wid * _PER_W
    pltpu.sync_copy(c_hbm, c_v)
    c0 = c_v[pl.ds(0, 16)]
    c1 = c_v[pl.ds(16, 16)]
    c2 = c_v[pl.ds(32, 16)]

    def chunk_body(k, carry):
        off = base + k * _CHUNK
        pltpu.sync_copy(e0_hbm.at[pl.ds(off, _CHUNK)], b0)
        pltpu.sync_copy(e1_hbm.at[pl.ds(off, _CHUNK)], b1)
        pltpu.sync_copy(e2_hbm.at[pl.ds(off, _CHUNK)], b2)

        def vec_body(i, carry2):
            s = i * 16
            ob[pl.ds(s, 16)] = (
                b0[pl.ds(s, 16)] * c0
                + b1[pl.ds(s, 16)] * c1
                + b2[pl.ds(s, 16)] * c2
            )
            return carry2

        lax.fori_loop(0, _CHUNK // 16, vec_body, 0)
        pltpu.sync_copy(ob, out_hbm.at[pl.ds(off, _CHUNK)])
        return carry

    lax.fori_loop(0, _NCHUNK, chunk_body, 0)


def kernel(embed_e0, embed_e1, embed_e2, coef_e0, coef_e1, coef_e2):
    coefs = jnp.concatenate(
        [
            jnp.broadcast_to(coef_e0, (16,)),
            jnp.broadcast_to(coef_e1, (16,)),
            jnp.broadcast_to(coef_e2, (16,)),
        ]
    )
    out = _sc_agg(
        embed_e0.reshape(_TOTAL),
        embed_e1.reshape(_TOTAL),
        embed_e2.reshape(_TOTAL),
        coefs,
    )
    return out.reshape(_N, _D)


# speedup vs baseline: 1.2658x; 1.2658x over previous
import functools

import jax
import jax.numpy as jnp
from jax import lax
from jax.experimental import pallas as pl
from jax.experimental.pallas import tpu as pltpu
from jax.experimental.pallas import tpu_sc as plsc

_N = 100000
_D = 128
_TOTAL = _N * _D
_NW = 32
_PER_W = _TOTAL // _NW
_CHUNK = 16000
_NCHUNK = _PER_W // _CHUNK

_mesh = plsc.VectorSubcoreMesh(core_axis_name="c", subcore_axis_name="s")


@functools.partial(
    pl.kernel,
    mesh=_mesh,
    out_type=jax.ShapeDtypeStruct((_TOTAL,), jnp.float32),
    scratch_types=[
        pltpu.VMEM((48,), jnp.float32),
        pltpu.VMEM((2, _CHUNK), jnp.float32),
        pltpu.VMEM((2, _CHUNK), jnp.float32),
        pltpu.VMEM((2, _CHUNK), jnp.float32),
        pltpu.VMEM((2, _CHUNK), jnp.float32),
        pltpu.SemaphoreType.DMA,
        pltpu.SemaphoreType.DMA,
        pltpu.SemaphoreType.DMA,
        pltpu.SemaphoreType.DMA,
    ],
)
def _sc_agg(e0_hbm, e1_hbm, e2_hbm, c_hbm, out_hbm, c_v, b0, b1, b2, ob,
            sin0, sin1, sout0, sout1):
    wid = lax.axis_index("s") * 2 + lax.axis_index("c")
    base = wid * _PER_W
    pltpu.sync_copy(c_hbm, c_v)
    c0 = c_v[pl.ds(0, 16)]
    c1 = c_v[pl.ds(16, 16)]
    c2 = c_v[pl.ds(32, 16)]
    sin = (sin0, sin1)
    sout = (sout0, sout1)

    def issue_in(k, j):
        off = base + k * _CHUNK
        pltpu.async_copy(e0_hbm.at[pl.ds(off, _CHUNK)], b0.at[j], sin[j])
        pltpu.async_copy(e1_hbm.at[pl.ds(off, _CHUNK)], b1.at[j], sin[j])
        pltpu.async_copy(e2_hbm.at[pl.ds(off, _CHUNK)], b2.at[j], sin[j])

    def wait_in(j):
        pltpu.make_async_copy(e0_hbm.at[pl.ds(0, _CHUNK)], b0.at[j], sin[j]).wait()
        pltpu.make_async_copy(e1_hbm.at[pl.ds(0, _CHUNK)], b1.at[j], sin[j]).wait()
        pltpu.make_async_copy(e2_hbm.at[pl.ds(0, _CHUNK)], b2.at[j], sin[j]).wait()

    issue_in(0, 0)
    for k in range(_NCHUNK):
        j = k % 2
        if k + 1 < _NCHUNK:
            issue_in(k + 1, 1 - j)
        wait_in(j)
        if k >= 2:
            pltpu.make_async_copy(
                ob.at[j], out_hbm.at[pl.ds(0, _CHUNK)], sout[j]
            ).wait()

        def vec_body(i, carry2, _j=j):
            s = i * 16
            ob[_j, pl.ds(s, 16)] = (
                b0[_j, pl.ds(s, 16)] * c0
                + b1[_j, pl.ds(s, 16)] * c1
                + b2[_j, pl.ds(s, 16)] * c2
            )
            return carry2

        lax.fori_loop(0, _CHUNK // 16, vec_body, 0)
        off = base + k * _CHUNK
        pltpu.async_copy(ob.at[j], out_hbm.at[pl.ds(off, _CHUNK)], sout[j])
    for j in (1, 0):
        pltpu.make_async_copy(
            ob.at[j], out_hbm.at[pl.ds(0, _CHUNK)], sout[j]
        ).wait()


def kernel(embed_e0, embed_e1, embed_e2, coef_e0, coef_e1, coef_e2):
    coefs = jnp.concatenate(
        [
            jnp.broadcast_to(coef_e0, (16,)),
            jnp.broadcast_to(coef_e1, (16,)),
            jnp.broadcast_to(coef_e2, (16,)),
        ]
    )
    out = _sc_agg(
        embed_e0.reshape(_TOTAL),
        embed_e1.reshape(_TOTAL),
        embed_e2.reshape(_TOTAL),
        coefs,
    )
    return out.reshape(_N, _D)


# speedup vs baseline: 1.6518x; 1.3049x over previous
import functools

import jax
import jax.numpy as jnp
from jax import lax
from jax.experimental import pallas as pl
from jax.experimental.pallas import tpu as pltpu
from jax.experimental.pallas import tpu_sc as plsc

_N = 100000
_D = 128
_TOTAL = _N * _D
_NW = 32
_PER_W = _TOTAL // _NW
_CHUNK = 16000
_NCHUNK = _PER_W // _CHUNK

_mesh = plsc.VectorSubcoreMesh(core_axis_name="c", subcore_axis_name="s")


@functools.partial(
    pl.kernel,
    mesh=_mesh,
    out_type=jax.ShapeDtypeStruct((_TOTAL,), jnp.float32),
    scratch_types=[
        pltpu.VMEM((48,), jnp.float32),
        pltpu.VMEM((2, _CHUNK), jnp.float32),
        pltpu.VMEM((2, _CHUNK), jnp.float32),
        pltpu.VMEM((2, _CHUNK), jnp.float32),
        pltpu.VMEM((2, _CHUNK), jnp.float32),
        pltpu.SemaphoreType.DMA,
        pltpu.SemaphoreType.DMA,
        pltpu.SemaphoreType.DMA,
        pltpu.SemaphoreType.DMA,
    ],
)
def _sc_agg(e0_hbm, e1_hbm, e2_hbm, c_hbm, out_hbm, c_v, b0, b1, b2, ob,
            sin0, sin1, sout0, sout1):
    wid = lax.axis_index("s") * 2 + lax.axis_index("c")
    base = wid * _PER_W
    pltpu.sync_copy(c_hbm, c_v)
    c0 = c_v[pl.ds(0, 16)]
    c1 = c_v[pl.ds(16, 16)]
    c2 = c_v[pl.ds(32, 16)]
    sin = (sin0, sin1)
    sout = (sout0, sout1)

    def issue_in(k, j):
        off = base + k * _CHUNK
        pltpu.async_copy(e0_hbm.at[pl.ds(off, _CHUNK)], b0.at[j], sin[j])
        pltpu.async_copy(e1_hbm.at[pl.ds(off, _CHUNK)], b1.at[j], sin[j])
        pltpu.async_copy(e2_hbm.at[pl.ds(off, _CHUNK)], b2.at[j], sin[j])

    def wait_in(j):
        pltpu.make_async_copy(e0_hbm.at[pl.ds(0, _CHUNK)], b0.at[j], sin[j]).wait()
        pltpu.make_async_copy(e1_hbm.at[pl.ds(0, _CHUNK)], b1.at[j], sin[j]).wait()
        pltpu.make_async_copy(e2_hbm.at[pl.ds(0, _CHUNK)], b2.at[j], sin[j]).wait()

    issue_in(0, 0)
    for k in range(_NCHUNK):
        j = k % 2
        if k + 1 < _NCHUNK:
            issue_in(k + 1, 1 - j)
        wait_in(j)
        if k >= 2:
            pltpu.make_async_copy(
                ob.at[j], out_hbm.at[pl.ds(0, _CHUNK)], sout[j]
            ).wait()

        @plsc.parallel_loop(0, _CHUNK, step=16, unroll=8)
        def _vec_body(s, _j=j):
            ob[_j, pl.ds(s, 16)] = (
                b0[_j, pl.ds(s, 16)] * c0
                + b1[_j, pl.ds(s, 16)] * c1
                + b2[_j, pl.ds(s, 16)] * c2
            )
        off = base + k * _CHUNK
        pltpu.async_copy(ob.at[j], out_hbm.at[pl.ds(off, _CHUNK)], sout[j])
    for j in (1, 0):
        pltpu.make_async_copy(
            ob.at[j], out_hbm.at[pl.ds(0, _CHUNK)], sout[j]
        ).wait()


def kernel(embed_e0, embed_e1, embed_e2, coef_e0, coef_e1, coef_e2):
    coefs = jnp.concatenate(
        [
            jnp.broadcast_to(coef_e0, (16,)),
            jnp.broadcast_to(coef_e1, (16,)),
            jnp.broadcast_to(coef_e2, (16,)),
        ]
    )
    out = _sc_agg(
        embed_e0.reshape(_TOTAL),
        embed_e1.reshape(_TOTAL),
        embed_e2.reshape(_TOTAL),
        coefs,
    )
    return out.reshape(_N, _D)


# speedup vs baseline: 2.3019x; 1.3936x over previous
import functools

import jax
import jax.numpy as jnp
from jax import lax
from jax.experimental import pallas as pl
from jax.experimental.pallas import tpu as pltpu
from jax.experimental.pallas import tpu_sc as plsc

_N = 100000
_D = 128
_N_SC = 32000
_N_TC = _N - _N_SC
_B = 4000
_NW = 32
_PER_W = _N_SC * _D // _NW
_CHUNK = 16000
_NCHUNK = _PER_W // _CHUNK
_SC_BASE = _N_TC * _D

_mesh = plsc.VectorSubcoreMesh(core_axis_name="c", subcore_axis_name="s")


def _tc_body(c0_ref, c1_ref, c2_ref, e0_ref, e1_ref, e2_ref, o_ref):
    o_ref[...] = (
        e0_ref[...] * c0_ref[0]
        + e1_ref[...] * c1_ref[0]
        + e2_ref[...] * c2_ref[0]
    )


def _tc_main(coef_e0, coef_e1, coef_e2, embed_e0, embed_e1, embed_e2):
    blk = pl.BlockSpec((_B, _D), lambda i: (i, 0))
    return pl.pallas_call(
        _tc_body,
        grid=(_N_TC // _B,),
        in_specs=[
            pl.BlockSpec(memory_space=pltpu.SMEM),
            pl.BlockSpec(memory_space=pltpu.SMEM),
            pl.BlockSpec(memory_space=pltpu.SMEM),
            blk,
            blk,
            blk,
        ],
        out_specs=blk,
        out_shape=jax.ShapeDtypeStruct((_N, _D), jnp.float32),
        compiler_params=pltpu.CompilerParams(
            dimension_semantics=("arbitrary",),
        ),
    )(coef_e0, coef_e1, coef_e2, embed_e0, embed_e1, embed_e2)


@functools.partial(
    pl.kernel,
    mesh=_mesh,
    out_type=jax.ShapeDtypeStruct((_N_SC * _D,), jnp.float32),
    scratch_types=[
        pltpu.VMEM((48,), jnp.float32),
        pltpu.VMEM((2, _CHUNK), jnp.float32),
        pltpu.VMEM((2, _CHUNK), jnp.float32),
        pltpu.VMEM((2, _CHUNK), jnp.float32),
        pltpu.VMEM((2, _CHUNK), jnp.float32),
        pltpu.SemaphoreType.DMA,
        pltpu.SemaphoreType.DMA,
        pltpu.SemaphoreType.DMA,
        pltpu.SemaphoreType.DMA,
    ],
)
def _sc_agg(e0_hbm, e1_hbm, e2_hbm, c_hbm, out_hbm, c_v, b0, b1, b2, ob,
            sin0, sin1, sout0, sout1):
    wid = lax.axis_index("s") * 2 + lax.axis_index("c")
    in_base = _SC_BASE + wid * _PER_W
    out_base = wid * _PER_W
    pltpu.sync_copy(c_hbm, c_v)
    c0 = c_v[pl.ds(0, 16)]
    c1 = c_v[pl.ds(16, 16)]
    c2 = c_v[pl.ds(32, 16)]
    sin = (sin0, sin1)
    sout = (sout0, sout1)

    def issue_in(k, j):
        off = in_base + k * _CHUNK
        pltpu.async_copy(e0_hbm.at[pl.ds(off, _CHUNK)], b0.at[j], sin[j])
        pltpu.async_copy(e1_hbm.at[pl.ds(off, _CHUNK)], b1.at[j], sin[j])
        pltpu.async_copy(e2_hbm.at[pl.ds(off, _CHUNK)], b2.at[j], sin[j])

    def wait_in(j):
        pltpu.make_async_copy(e0_hbm.at[pl.ds(0, _CHUNK)], b0.at[j], sin[j]).wait()
        pltpu.make_async_copy(e1_hbm.at[pl.ds(0, _CHUNK)], b1.at[j], sin[j]).wait()
        pltpu.make_async_copy(e2_hbm.at[pl.ds(0, _CHUNK)], b2.at[j], sin[j]).wait()

    issue_in(0, 0)
    for k in range(_NCHUNK):
        j = k % 2
        if k + 1 < _NCHUNK:
            issue_in(k + 1, 1 - j)
        wait_in(j)
        if k >= 2:
            pltpu.make_async_copy(
                ob.at[j], out_hbm.at[pl.ds(0, _CHUNK)], sout[j]
            ).wait()

        @plsc.parallel_loop(0, _CHUNK, step=16, unroll=8)
        def _vec_body(s, _j=j):
            ob[_j, pl.ds(s, 16)] = (
                b0[_j, pl.ds(s, 16)] * c0
                + b1[_j, pl.ds(s, 16)] * c1
                + b2[_j, pl.ds(s, 16)] * c2
            )

        pltpu.async_copy(
            ob.at[j], out_hbm.at[pl.ds(out_base + k * _CHUNK, _CHUNK)], sout[j]
        )
    for j in (1, 0):
        pltpu.make_async_copy(
            ob.at[j], out_hbm.at[pl.ds(0, _CHUNK)], sout[j]
        ).wait()


def _merge_body(full_ref, sc_ref, o_ref):
    del full_ref
    o_ref[...] = sc_ref[...]


def _merge(out_full, sc_rows):
    return pl.pallas_call(
        _merge_body,
        grid=(_N_SC // _B,),
        in_specs=[
            pl.BlockSpec(memory_space=pl.ANY),
            pl.BlockSpec((_B, _D), lambda i: (i, 0)),
        ],
        out_specs=pl.BlockSpec((_B, _D), lambda i: (i + _N_TC // _B, 0)),
        out_shape=jax.ShapeDtypeStruct((_N, _D), jnp.float32),
        input_output_aliases={0: 0},
        compiler_params=pltpu.CompilerParams(
            dimension_semantics=("arbitrary",),
        ),
    )(out_full, sc_rows)


def kernel(embed_e0, embed_e1, embed_e2, coef_e0, coef_e1, coef_e2):
    coefs = jnp.concatenate(
        [
            jnp.broadcast_to(coef_e0, (16,)),
            jnp.broadcast_to(coef_e1, (16,)),
            jnp.broadcast_to(coef_e2, (16,)),
        ]
    )
    sc_flat = _sc_agg(
        embed_e0.reshape(_N * _D),
        embed_e1.reshape(_N * _D),
        embed_e2.reshape(_N * _D),
        coefs,
    )
    out_full = _tc_main(
        coef_e0, coef_e1, coef_e2, embed_e0, embed_e1, embed_e2
    )
    return _merge(out_full, sc_flat.reshape(_N_SC, _D))


# speedup vs baseline: 3.5298x; 1.5334x over previous
import jax
import jax.numpy as jnp
from jax.experimental import pallas as pl
from jax.experimental.pallas import tpu as pltpu


def _agg_body(c0_ref, c1_ref, c2_ref, e0_ref, e1_ref, e2_ref, o_ref):
    o_ref[...] = (
        e0_ref[...] * c0_ref[0]
        + e1_ref[...] * c1_ref[0]
        + e2_ref[...] * c2_ref[0]
    )


def kernel(embed_e0, embed_e1, embed_e2, coef_e0, coef_e1, coef_e2):
    N, D = embed_e0.shape
    B = 8000
    blk = pl.BlockSpec((B, D), lambda i: (i, 0))
    return pl.pallas_call(
        _agg_body,
        grid=(N // B,),
        in_specs=[
            pl.BlockSpec(memory_space=pltpu.SMEM),
            pl.BlockSpec(memory_space=pltpu.SMEM),
            pl.BlockSpec(memory_space=pltpu.SMEM),
            blk,
            blk,
            blk,
        ],
        out_specs=blk,
        out_shape=jax.ShapeDtypeStruct((N, D), embed_e0.dtype),
        compiler_params=pltpu.CompilerParams(
            dimension_semantics=("arbitrary",),
        ),
    )(coef_e0, coef_e1, coef_e2, embed_e0, embed_e1, embed_e2)
